# baseline (speedup 1.0000x reference)
import jax
import jax.numpy as jnp
from jax import lax
from jax.experimental import pallas as pl

_BN = 400


def _body(x_ref, wc_ref, bc_ref, wr_ref, br_ref, cls_ref, reg_ref):
    xm = jnp.sum(x_ref[...], axis=0)
    dn = (((1,), (1,)), ((), ()))
    cls_ref[...] = (
        lax.dot_general(xm, wc_ref[...], dn, preferred_element_type=jnp.float32)
        + bc_ref[...]
    )
    reg_ref[...] = (
        lax.dot_general(xm, wr_ref[...], dn, preferred_element_type=jnp.float32)
        + br_ref[...]
    )


def kernel(x, W_cls, b_cls, W_reg, b_reg):
    n, c, rh, rw = x.shape
    s = rh * rw
    k1 = W_cls.shape[0]
    k2 = W_reg.shape[0]
    x4 = x.transpose(2, 3, 0, 1).reshape(s, n, c)
    wc = W_cls * (1.0 / s)
    wr = W_reg * (1.0 / s)
    bc2 = b_cls.reshape(1, k1)
    br2 = b_reg.reshape(1, k2)
    cls, reg = pl.pallas_call(
        _body,
        grid=(n // _BN,),
        in_specs=[
            pl.BlockSpec((s, _BN, c), lambda i: (0, i, 0)),
            pl.BlockSpec((k1, c), lambda i: (0, 0)),
            pl.BlockSpec((1, k1), lambda i: (0, 0)),
            pl.BlockSpec((k2, c), lambda i: (0, 0)),
            pl.BlockSpec((1, k2), lambda i: (0, 0)),
        ],
        out_specs=[
            pl.BlockSpec((_BN, k1), lambda i: (i, 0)),
            pl.BlockSpec((_BN, k2), lambda i: (i, 0)),
        ],
        out_shape=[
            jax.ShapeDtypeStruct((n, k1), jnp.float32),
            jax.ShapeDtypeStruct((n, k2), jnp.float32),
        ],
    )(x4, wc, bc2, wr, br2)
    return (cls, reg)


# speedup vs baseline: 1.1321x; 1.1321x over previous
import jax
import jax.numpy as jnp
from jax import lax
from jax.experimental import pallas as pl

_BN = 128


def _body(x_ref, wc_ref, bc_ref, wr_ref, br_ref, cls_ref, reg_ref):
    s = x_ref.shape[0]
    xm = jnp.sum(x_ref[...], axis=0) * (1.0 / s)
    dn = (((1,), (1,)), ((), ()))
    cls_ref[...] = (
        lax.dot_general(wc_ref[...], xm, dn, preferred_element_type=jnp.float32)
        + bc_ref[...]
    )
    reg_ref[...] = (
        lax.dot_general(wr_ref[...], xm, dn, preferred_element_type=jnp.float32)
        + br_ref[...]
    )


def kernel(x, W_cls, b_cls, W_reg, b_reg):
    n, c, rh, rw = x.shape
    s = rh * rw
    k1 = W_cls.shape[0]
    k2 = W_reg.shape[0]
    x4 = x.transpose(2, 3, 0, 1).reshape(s, n, c)
    bc2 = b_cls.reshape(k1, 1)
    br2 = b_reg.reshape(k2, 1)
    grid = (n + _BN - 1) // _BN
    cls_t, reg_t = pl.pallas_call(
        _body,
        grid=(grid,),
        in_specs=[
            pl.BlockSpec((s, _BN, c), lambda i: (0, i, 0)),
            pl.BlockSpec((k1, c), lambda i: (0, 0)),
            pl.BlockSpec((k1, 1), lambda i: (0, 0)),
            pl.BlockSpec((k2, c), lambda i: (0, 0)),
            pl.BlockSpec((k2, 1), lambda i: (0, 0)),
        ],
        out_specs=[
            pl.BlockSpec((k1, _BN), lambda i: (0, i)),
            pl.BlockSpec((k2, _BN), lambda i: (0, i)),
        ],
        out_shape=[
            jax.ShapeDtypeStruct((k1, n), jnp.float32),
            jax.ShapeDtypeStruct((k2, n), jnp.float32),
        ],
    )(x4, W_cls, bc2, W_reg, br2)
    return (cls_t.T, reg_t.T)


# speedup vs baseline: 1.1326x; 1.0005x over previous
import jax
import jax.numpy as jnp
from jax import lax
from jax.experimental import pallas as pl

_BN = 256


def _body(x_ref, wc_ref, bc_ref, wr_ref, br_ref, cls_ref, reg_ref):
    s = x_ref.shape[0]
    xm = jnp.sum(x_ref[...], axis=0) * (1.0 / s)
    dn = (((1,), (1,)), ((), ()))
    cls_ref[...] = (
        lax.dot_general(wc_ref[...], xm, dn, preferred_element_type=jnp.float32)
        + bc_ref[...]
    )
    reg_ref[...] = (
        lax.dot_general(wr_ref[...], xm, dn, preferred_element_type=jnp.float32)
        + br_ref[...]
    )


def kernel(x, W_cls, b_cls, W_reg, b_reg):
    n, c, rh, rw = x.shape
    s = rh * rw
    k1 = W_cls.shape[0]
    k2 = W_reg.shape[0]
    x4 = x.transpose(2, 3, 0, 1).reshape(s, n, c)
    bc2 = b_cls.reshape(k1, 1)
    br2 = b_reg.reshape(k2, 1)
    grid = (n + _BN - 1) // _BN
    cls_t, reg_t = pl.pallas_call(
        _body,
        grid=(grid,),
        in_specs=[
            pl.BlockSpec((s, _BN, c), lambda i: (0, i, 0)),
            pl.BlockSpec((k1, c), lambda i: (0, 0)),
            pl.BlockSpec((k1, 1), lambda i: (0, 0)),
            pl.BlockSpec((k2, c), lambda i: (0, 0)),
            pl.BlockSpec((k2, 1), lambda i: (0, 0)),
        ],
        out_specs=[
            pl.BlockSpec((k1, _BN), lambda i: (0, i)),
            pl.BlockSpec((k2, _BN), lambda i: (0, i)),
        ],
        out_shape=[
            jax.ShapeDtypeStruct((k1, n), jnp.float32),
            jax.ShapeDtypeStruct((k2, n), jnp.float32),
        ],
    )(x4, W_cls, bc2, W_reg, br2)
    return (cls_t.T, reg_t.T)
